# baseline (speedup 1.0000x reference)
import jax
import jax.numpy as jnp
from jax import lax
from jax.experimental import pallas as pl
from jax.experimental.pallas import tpu as pltpu
from jax.experimental.pallas import tpu_sc as plsc
import functools

_D = 256
_H = 64
_W = 64
_S = _H * _W
_NP = 32
_NB = 8
_NJ = _NP + _NB


def _pool_body(mask_ref, a_ref):
    f32 = jnp.float32
    B = a_ref.shape[0]
    c = lax.broadcasted_iota(jnp.int32, (4 * _W, 2 * _W), 0)
    vx = lax.broadcasted_iota(jnp.int32, (4 * _W, 2 * _W), 1)
    v_ = vx // _W
    x_ = vx % _W
    Ccat = ((c == 4 * x_ + v_) | (c == 4 * x_ + v_ + 2)).astype(f32)
    Tc = jnp.dot(mask_ref[...], Ccat, preferred_element_type=f32)
    r = lax.broadcasted_iota(jnp.int32, (2 * _H, 4 * _H), 1)
    uy = lax.broadcasted_iota(jnp.int32, (2 * _H, 4 * _H), 0)
    u_ = uy // _H
    y_ = uy % _H
    Rcat = ((r == 4 * y_ + u_) | (r == 4 * y_ + u_ + 2)).astype(f32)
    for b in range(B):
        U = jnp.dot(Rcat, Tc[b * 256:(b + 1) * 256, :],
                    preferred_element_type=f32) * 0.25
        for u in (0, 1):
            for v in (0, 1):
                a_ref[b, 2 * u + v] = U[u * _H:(u + 1) * _H,
                                        v * _W:(v + 1) * _W]
        a_ref[b, 4] = jnp.ones((_H, _W), f32)


def _winner_sc_body(pts_hbm, box_hbm, win_hbm, pts_v, box_v, win_v):
    i32 = jnp.int32
    nc = 2
    wid = lax.axis_index("s") * nc + lax.axis_index("c")
    pltpu.sync_copy(pts_hbm, pts_v)
    pltpu.sync_copy(box_hbm, box_v)
    base = wid * (_S // 32)
    lane = lax.broadcasted_iota(i32, (16,), 0)
    one = jnp.full((16,), 1, i32)
    zero = jnp.full((16,), 0, i32)
    pids = [base + k * 16 + lane for k in range(8)]
    ys = [lax.shift_right_logical(p, 6) for p in pids]
    xs = [p & (_W - 1) for p in pids]
    win = [jnp.full((16,), -1, i32) for _ in range(8)]

    def eq01(a, b):
        return one - jnp.minimum(one, jnp.abs(a - b))

    def ge01(a, b):
        return one - jnp.minimum(one, jnp.maximum(zero, b - a))

    def lt01(a, b):
        return jnp.minimum(one, jnp.maximum(zero, b - a))

    pchunks = [pts_v[pl.ds(c * 16, 16)] for c in range(_NP * 3 // 16)]
    bchunks = [box_v[pl.ds(c * 16, 16)] for c in range(_NB * 4 // 16)]

    def elem(chunks, flat_idx):
        return chunks[flat_idx // 16][flat_idx % 16]

    for j in range(_NP):
        x = elem(pchunks, 3 * j)
        y = elem(pchunks, 3 * j + 1)
        xi = jnp.clip(x.astype(i32), 0, _W - 1)
        yi = jnp.clip(y.astype(i32), 0, _H - 1)
        xv = jnp.minimum(jnp.maximum(x, 0.0), float(_W - 1))
        yv = jnp.minimum(jnp.maximum(y, 0.0), float(_H - 1))
        d = jnp.minimum(jnp.abs(x - xv) + jnp.abs(y - yv), 1.0)
        valid = 1 - jnp.minimum(1, (d * 1e9).astype(i32))
        for k in range(8):
            cov = eq01(ys[k], yi) * eq01(xs[k], xi) * valid
            win[k] = jnp.maximum(win[k], cov * (j + 1) - 1)
    for j in range(_NB):
        x1 = elem(bchunks, 4 * j).astype(i32)
        y1 = elem(bchunks, 4 * j + 1).astype(i32)
        x2 = elem(bchunks, 4 * j + 2).astype(i32)
        y2 = elem(bchunks, 4 * j + 3).astype(i32)
        for k in range(8):
            cov = (ge01(ys[k], y1) * lt01(ys[k], y2)
                   * ge01(xs[k], x1) * lt01(xs[k], x2))
            win[k] = jnp.maximum(win[k], cov * (j + _NP + 1) - 1)
    for k in range(8):
        win_v[pl.ds(k * 16, 16)] = win[k]
    pltpu.sync_copy(win_v, win_hbm.at[pl.ds(base, _S // 32)])


def _combine_body(a_ref, w5_ref, out_ref):
    f32 = jnp.float32
    for i in range(a_ref.shape[0]):
        out_ref[i] = jnp.dot(w5_ref[...], a_ref[i],
                             preferred_element_type=f32)


def _batch0_body(pts_s, box_s, win_ref, a_ref, ptT_ref, bW_ref, bb_ref,
                 w5_ref, prev_ref, out_ref):
    del prev_ref
    f32 = jnp.float32
    m = jnp.dot(w5_ref[...], a_ref[0], preferred_element_type=f32)
    if True:
        winner = win_ref[...]
        cols = []
        for i in range(_NP):
            l = pts_s[i, 2]
            li = jnp.clip(l.astype(jnp.int32), 0, 2)
            colp = jnp.where(li == 0, ptT_ref[:, 0:1],
                             jnp.where(li == 1, ptT_ref[:, 1:2],
                                       ptT_ref[:, 2:3]))
            cols.append(colp)
        for i in range(_NB):
            bcol = (bb_ref[...] + bW_ref[:, 0:1] * box_s[i, 0]
                    + bW_ref[:, 1:2] * box_s[i, 1]
                    + bW_ref[:, 2:3] * box_s[i, 2]
                    + bW_ref[:, 3:4] * box_s[i, 3])
            cols.append(bcol)
        ET = jnp.concatenate(cols, axis=1)
        jidx = lax.broadcasted_iota(jnp.int32, (_NJ, _S), 0)
        S = (jidx == winner).astype(f32)
        out_ref[0] = m + jnp.dot(ET, S, preferred_element_type=f32)


def kernel(points, boxes, masks, point_table, box_W, box_b, conv_w, conv_b,
           no_mask_embed):
    del no_mask_embed
    B = points.shape[0]
    pts0 = points[0]
    box0 = boxes[0]
    ptT = point_table.T
    bb = box_b.reshape(_D, 1)
    w4 = conv_w.reshape(_D, 4)
    cb = conv_b.reshape(_D, 1)
    w5 = jnp.concatenate([w4, cb], axis=1)

    mesh = plsc.VectorSubcoreMesh(core_axis_name="c", subcore_axis_name="s")
    winner = pl.kernel(
        _winner_sc_body,
        mesh=mesh,
        out_type=jax.ShapeDtypeStruct((_S,), jnp.int32),
        scratch_types=[
            pltpu.VMEM((_NP * 3,), jnp.float32),
            pltpu.VMEM((_NB * 4,), jnp.float32),
            pltpu.VMEM((_S // 32,), jnp.int32),
        ],
    )(pts0.reshape(_NP * 3), box0.reshape(_NB * 4))
    win2 = winner.reshape(1, _S)

    masks_flat = masks.reshape(B * 4 * _H, 4 * _W)
    a4 = pl.pallas_call(
        _pool_body,
        grid=(1,),
        in_specs=[pl.BlockSpec((B * 4 * _H, 4 * _W), lambda i: (0, 0))],
        out_specs=pl.BlockSpec((B, 5, _H, _W), lambda i: (0, 0, 0, 0)),
        out_shape=jax.ShapeDtypeStruct((B, 5, _H, _W), jnp.float32),
        interpret=_INTERPRET,
    )(masks_flat)
    a_flat = a4.reshape(B, 5, _S)

    GB = 4
    out1 = pl.pallas_call(
        _combine_body,
        grid=(B // GB,),
        in_specs=[
            pl.BlockSpec((GB, 5, _S), lambda b: (b, 0, 0)),
            pl.BlockSpec((_D, 5), lambda b: (0, 0)),
        ],
        out_specs=pl.BlockSpec((GB, _D, _S), lambda b: (b, 0, 0)),
        out_shape=jax.ShapeDtypeStruct((B, _D, _S), jnp.float32),
        interpret=_INTERPRET,
    )(a_flat, w5)

    out = pl.pallas_call(
        _batch0_body,
        grid=(1,),
        in_specs=[
            pl.BlockSpec(memory_space=pltpu.SMEM),
            pl.BlockSpec(memory_space=pltpu.SMEM),
            pl.BlockSpec((1, _S), lambda i: (0, 0)),
            pl.BlockSpec((1, 5, _S), lambda i: (0, 0, 0)),
            pl.BlockSpec((_D, 3), lambda i: (0, 0)),
            pl.BlockSpec((_D, 4), lambda i: (0, 0)),
            pl.BlockSpec((_D, 1), lambda i: (0, 0)),
            pl.BlockSpec((_D, 5), lambda i: (0, 0)),
            pl.BlockSpec(memory_space=pl.ANY),
        ],
        out_specs=pl.BlockSpec((1, _D, _S), lambda i: (0, 0, 0)),
        out_shape=jax.ShapeDtypeStruct((B, _D, _S), jnp.float32),
        input_output_aliases={8: 0},
        interpret=_INTERPRET,
    )(pts0, box0, win2, a_flat, ptT, box_W, bb, w5, out1)
    return out.reshape(B, _D, _H, _W)


_INTERPRET = False


# speedup vs baseline: 1.0355x; 1.0355x over previous
import jax
import jax.numpy as jnp
from jax import lax
from jax.experimental import pallas as pl
from jax.experimental.pallas import tpu as pltpu
from jax.experimental.pallas import tpu_sc as plsc
import functools

_D = 256
_H = 64
_W = 64
_S = _H * _W
_NP = 32
_NB = 8
_NJ = _NP + _NB


def _pool_body(mask_ref, a_ref):
    f32 = jnp.float32
    B = a_ref.shape[0]
    c = lax.broadcasted_iota(jnp.int32, (4 * _W, 2 * _W), 0)
    vx = lax.broadcasted_iota(jnp.int32, (4 * _W, 2 * _W), 1)
    v_ = vx // _W
    x_ = vx % _W
    Ccat = ((c == 4 * x_ + v_) | (c == 4 * x_ + v_ + 2)).astype(f32)
    Tc = jnp.dot(mask_ref[...], Ccat, preferred_element_type=f32)
    r = lax.broadcasted_iota(jnp.int32, (2 * _H, 4 * _H), 1)
    uy = lax.broadcasted_iota(jnp.int32, (2 * _H, 4 * _H), 0)
    u_ = uy // _H
    y_ = uy % _H
    Rcat = ((r == 4 * y_ + u_) | (r == 4 * y_ + u_ + 2)).astype(f32)
    for b in range(B):
        U = jnp.dot(Rcat, Tc[b * 256:(b + 1) * 256, :],
                    preferred_element_type=f32) * 0.25
        for u in (0, 1):
            for v in (0, 1):
                a_ref[b, 2 * u + v] = U[u * _H:(u + 1) * _H,
                                        v * _W:(v + 1) * _W]
        a_ref[b, 4] = jnp.ones((_H, _W), f32)


def _winner_sc_body(pts_hbm, box_hbm, win_hbm, pts_v, box_v, win_v):
    i32 = jnp.int32
    nc = 2
    wid = lax.axis_index("s") * nc + lax.axis_index("c")
    pltpu.sync_copy(pts_hbm, pts_v)
    pltpu.sync_copy(box_hbm, box_v)
    base = wid * (_S // 32)
    lane = lax.broadcasted_iota(i32, (16,), 0)
    one = jnp.full((16,), 1, i32)
    zero = jnp.full((16,), 0, i32)
    pids = [base + k * 16 + lane for k in range(8)]
    ys = [lax.shift_right_logical(p, 6) for p in pids]
    xs = [p & (_W - 1) for p in pids]
    win = [jnp.full((16,), -1, i32) for _ in range(8)]

    def eq01(a, b):
        return one - jnp.minimum(one, jnp.abs(a - b))

    def ge01(a, b):
        return one - jnp.minimum(one, jnp.maximum(zero, b - a))

    def lt01(a, b):
        return jnp.minimum(one, jnp.maximum(zero, b - a))

    pchunks = [pts_v[pl.ds(c * 16, 16)] for c in range(_NP * 3 // 16)]
    bchunks = [box_v[pl.ds(c * 16, 16)] for c in range(_NB * 4 // 16)]

    def elem(chunks, flat_idx):
        return chunks[flat_idx // 16][flat_idx % 16]

    for j in range(_NP):
        x = elem(pchunks, 3 * j)
        y = elem(pchunks, 3 * j + 1)
        xi = jnp.clip(x.astype(i32), 0, _W - 1)
        yi = jnp.clip(y.astype(i32), 0, _H - 1)
        xv = jnp.minimum(jnp.maximum(x, 0.0), float(_W - 1))
        yv = jnp.minimum(jnp.maximum(y, 0.0), float(_H - 1))
        d = jnp.minimum(jnp.abs(x - xv) + jnp.abs(y - yv), 1.0)
        valid = 1 - jnp.minimum(1, (d * 1e9).astype(i32))
        for k in range(8):
            cov = eq01(ys[k], yi) * eq01(xs[k], xi) * valid
            win[k] = jnp.maximum(win[k], cov * (j + 1) - 1)
    for j in range(_NB):
        x1 = elem(bchunks, 4 * j).astype(i32)
        y1 = elem(bchunks, 4 * j + 1).astype(i32)
        x2 = elem(bchunks, 4 * j + 2).astype(i32)
        y2 = elem(bchunks, 4 * j + 3).astype(i32)
        for k in range(8):
            cov = (ge01(ys[k], y1) * lt01(ys[k], y2)
                   * ge01(xs[k], x1) * lt01(xs[k], x2))
            win[k] = jnp.maximum(win[k], cov * (j + _NP + 1) - 1)
    for k in range(8):
        win_v[pl.ds(k * 16, 16)] = win[k]
    pltpu.sync_copy(win_v, win_hbm.at[pl.ds(base, _S // 32)])


def _combine_body(pts_s, box_s, win_ref, a_ref, ptT_ref, bW_ref, bb_ref,
                  w5_ref, out_ref):
    b = pl.program_id(0)
    f32 = jnp.float32
    for i in range(1, a_ref.shape[0]):
        out_ref[i] = jnp.dot(w5_ref[...], a_ref[i],
                             preferred_element_type=f32)
    m = jnp.dot(w5_ref[...], a_ref[0], preferred_element_type=f32)

    @pl.when(b == 0)
    def _scatter():
        winner = win_ref[...]
        cols = []
        for i in range(_NP):
            l = pts_s[i, 2]
            li = jnp.clip(l.astype(jnp.int32), 0, 2)
            colp = jnp.where(li == 0, ptT_ref[:, 0:1],
                             jnp.where(li == 1, ptT_ref[:, 1:2],
                                       ptT_ref[:, 2:3]))
            cols.append(colp)
        for i in range(_NB):
            bcol = (bb_ref[...] + bW_ref[:, 0:1] * box_s[i, 0]
                    + bW_ref[:, 1:2] * box_s[i, 1]
                    + bW_ref[:, 2:3] * box_s[i, 2]
                    + bW_ref[:, 3:4] * box_s[i, 3])
            cols.append(bcol)
        ET = jnp.concatenate(cols, axis=1)
        jidx = lax.broadcasted_iota(jnp.int32, (_NJ, _S), 0)
        S = (jidx == winner).astype(f32)
        out_ref[0] = m + jnp.dot(ET, S, preferred_element_type=f32)

    @pl.when(b != 0)
    def _plain():
        out_ref[0] = m


def kernel(points, boxes, masks, point_table, box_W, box_b, conv_w, conv_b,
           no_mask_embed):
    del no_mask_embed
    B = points.shape[0]
    pts0 = points[0]
    box0 = boxes[0]
    ptT = point_table.T
    bb = box_b.reshape(_D, 1)
    w4 = conv_w.reshape(_D, 4)
    cb = conv_b.reshape(_D, 1)
    w5 = jnp.concatenate([w4, cb], axis=1)

    mesh = plsc.VectorSubcoreMesh(core_axis_name="c", subcore_axis_name="s")
    winner = pl.kernel(
        _winner_sc_body,
        mesh=mesh,
        out_type=jax.ShapeDtypeStruct((_S,), jnp.int32),
        scratch_types=[
            pltpu.VMEM((_NP * 3,), jnp.float32),
            pltpu.VMEM((_NB * 4,), jnp.float32),
            pltpu.VMEM((_S // 32,), jnp.int32),
        ],
    )(pts0.reshape(_NP * 3), box0.reshape(_NB * 4))
    win2 = winner.reshape(1, _S)

    masks_flat = masks.reshape(B * 4 * _H, 4 * _W)
    a4 = pl.pallas_call(
        _pool_body,
        grid=(1,),
        in_specs=[pl.BlockSpec((B * 4 * _H, 4 * _W), lambda i: (0, 0))],
        out_specs=pl.BlockSpec((B, 5, _H, _W), lambda i: (0, 0, 0, 0)),
        out_shape=jax.ShapeDtypeStruct((B, 5, _H, _W), jnp.float32),
        interpret=_INTERPRET,
    )(masks_flat)
    a_flat = a4.reshape(B, 5, _S)

    GB = 4
    out = pl.pallas_call(
        _combine_body,
        grid=(B // GB,),
        in_specs=[
            pl.BlockSpec(memory_space=pltpu.SMEM),
            pl.BlockSpec(memory_space=pltpu.SMEM),
            pl.BlockSpec((1, _S), lambda b: (0, 0)),
            pl.BlockSpec((GB, 5, _S), lambda b: (b, 0, 0)),
            pl.BlockSpec((_D, 3), lambda b: (0, 0)),
            pl.BlockSpec((_D, 4), lambda b: (0, 0)),
            pl.BlockSpec((_D, 1), lambda b: (0, 0)),
            pl.BlockSpec((_D, 5), lambda b: (0, 0)),
        ],
        out_specs=pl.BlockSpec((GB, _D, _S), lambda b: (b, 0, 0)),
        out_shape=jax.ShapeDtypeStruct((B, _D, _S), jnp.float32),
        interpret=_INTERPRET,
    )(pts0, box0, win2, a_flat, ptT, box_W, bb, w5)
    return out.reshape(B, _D, _H, _W)


_INTERPRET = False
